# Initial kernel scaffold; baseline (speedup 1.0000x reference)
import jax
import jax.numpy as jnp
from jax.experimental import pallas as pl


def kernel(x, pos_table):
    raise NotImplementedError("write your pallas kernel here")


# speedup vs baseline: 1.4841x; 1.4841x over previous
import jax
import jax.numpy as jnp
from jax.experimental import pallas as pl


def _body(x_ref, p_ref, o_ref):
    o_ref[...] = x_ref[...] + p_ref[...]


def kernel(x, pos_table):
    B, N, D = x.shape
    BN = 512
    grid = (N // BN, B)
    return pl.pallas_call(
        _body,
        grid=grid,
        in_specs=[
            pl.BlockSpec((1, BN, D), lambda i, b: (b, i, 0)),
            pl.BlockSpec((BN, D), lambda i, b: (i, 0)),
        ],
        out_specs=pl.BlockSpec((1, BN, D), lambda i, b: (b, i, 0)),
        out_shape=jax.ShapeDtypeStruct((B, N, D), x.dtype),
    )(x, pos_table[:N])


# speedup vs baseline: 1.6670x; 1.1232x over previous
import jax
import jax.numpy as jnp
from jax.experimental import pallas as pl


def _body(x_ref, p_ref, o_ref):
    o_ref[...] = x_ref[...] + p_ref[...]


def kernel(x, pos_table):
    B, N, D = x.shape
    BN = 1024
    grid = (N // BN, B)
    return pl.pallas_call(
        _body,
        grid=grid,
        in_specs=[
            pl.BlockSpec((1, BN, D), lambda i, b: (b, i, 0)),
            pl.BlockSpec((BN, D), lambda i, b: (i, 0)),
        ],
        out_specs=pl.BlockSpec((1, BN, D), lambda i, b: (b, i, 0)),
        out_shape=jax.ShapeDtypeStruct((B, N, D), x.dtype),
    )(x, pos_table[:N])


# speedup vs baseline: 1.7315x; 1.0387x over previous
import jax
import jax.numpy as jnp
from jax.experimental import pallas as pl


def _body(x_ref, p_ref, o_ref):
    o_ref[...] = x_ref[...] + p_ref[...]


def kernel(x, pos_table):
    B, N, D = x.shape
    BN = 2048
    grid = (N // BN, B)
    return pl.pallas_call(
        _body,
        grid=grid,
        in_specs=[
            pl.BlockSpec((1, BN, D), lambda i, b: (b, i, 0)),
            pl.BlockSpec((BN, D), lambda i, b: (i, 0)),
        ],
        out_specs=pl.BlockSpec((1, BN, D), lambda i, b: (b, i, 0)),
        out_shape=jax.ShapeDtypeStruct((B, N, D), x.dtype),
    )(x, pos_table[:N])
